# speedup vs baseline: 4.7980x; 4.7980x over previous
import functools

import jax
import jax.numpy as jnp
from jax import lax
from jax.experimental import pallas as pl
from jax.experimental.pallas import tpu as pltpu
from jax.experimental.pallas import tpu_sc as plsc

HIDDEN = 128
CHUNK = 128
LANES = 16


def _sc_embed_sum(ix, iy, it, x_table, y_table, t_table, n_tokens, n_workers):
    per_worker = n_tokens // n_workers
    n_chunks = per_worker // CHUNK
    mesh = plsc.VectorSubcoreMesh(core_axis_name="c", subcore_axis_name="s")

    @functools.partial(
        pl.kernel,
        mesh=mesh,
        out_type=jax.ShapeDtypeStruct((n_tokens, HIDDEN), jnp.float32),
        scratch_types=[
            pltpu.VMEM((CHUNK,), jnp.int32),
            pltpu.VMEM((CHUNK,), jnp.int32),
            pltpu.VMEM((CHUNK,), jnp.int32),
            pltpu.VMEM((CHUNK, HIDDEN), jnp.float32),
            pltpu.VMEM((CHUNK, HIDDEN), jnp.float32),
            pltpu.VMEM((CHUNK, HIDDEN), jnp.float32),
            pltpu.SemaphoreType.DMA,
        ],
    )
    def body(ix_hbm, iy_hbm, it_hbm, x_hbm, y_hbm, t_hbm, out_hbm,
             ixv, iyv, itv, bx, by, bt, sem):
        wid = lax.axis_index("s") * 2 + lax.axis_index("c")
        w_base = wid * per_worker

        def chunk_step(k, _):
            base = w_base + k * CHUNK
            pltpu.sync_copy(ix_hbm.at[pl.ds(base, CHUNK)], ixv)
            pltpu.sync_copy(iy_hbm.at[pl.ds(base, CHUNK)], iyv)
            pltpu.sync_copy(it_hbm.at[pl.ds(base, CHUNK)], itv)
            cx = pltpu.async_copy(x_hbm.at[ixv], bx, sem)
            cy = pltpu.async_copy(y_hbm.at[iyv], by, sem)
            ct = pltpu.async_copy(t_hbm.at[itv], bt, sem)
            cx.wait()
            cy.wait()
            ct.wait()

            def row_step(r, _):
                for c in range(HIDDEN // LANES):
                    sl = pl.ds(c * LANES, LANES)
                    bx[r, sl] = bx[r, sl] + by[r, sl] + bt[r, sl]
                return 0

            lax.fori_loop(0, CHUNK, row_step, 0)
            pltpu.sync_copy(bx, out_hbm.at[pl.ds(base, CHUNK)])
            return 0

        lax.fori_loop(0, n_chunks, chunk_step, 0)

    return body(ix, iy, it, x_table, y_table, t_table)


def kernel(position_ids, x_table, y_table, time_table):
    b, s, _ = position_ids.shape
    n_tokens = b * s
    ids = position_ids.reshape(n_tokens, 3).astype(jnp.int32)
    ix = ids[:, 0]
    iy = ids[:, 1]
    it = ids[:, 2]
    out = _sc_embed_sum(ix, iy, it, x_table, y_table, time_table,
                        n_tokens, n_workers=32)
    return out.reshape(b, s, HIDDEN)


# speedup vs baseline: 7.0455x; 1.4684x over previous
import functools

import jax
import jax.numpy as jnp
from jax import lax
from jax.experimental import pallas as pl
from jax.experimental.pallas import tpu as pltpu
from jax.experimental.pallas import tpu_sc as plsc

HIDDEN = 128
CHUNK = 128
LANES = 16
N_WORKERS = 32


def _sc_embed_sum(ix, iy, it, x_table, y_table, t_table, n_tokens):
    per_worker = n_tokens // N_WORKERS
    n_chunks = per_worker // CHUNK
    mesh = plsc.VectorSubcoreMesh(core_axis_name="c", subcore_axis_name="s")

    @functools.partial(
        pl.kernel,
        mesh=mesh,
        out_type=jax.ShapeDtypeStruct((n_tokens, HIDDEN), jnp.float32),
        scratch_types=[
            pltpu.VMEM((n_chunks, CHUNK), jnp.int32),
            pltpu.VMEM((n_chunks, CHUNK), jnp.int32),
            pltpu.VMEM((n_chunks, CHUNK), jnp.int32),
            pltpu.VMEM((CHUNK, HIDDEN), jnp.float32),
            pltpu.VMEM((CHUNK, HIDDEN), jnp.float32),
            pltpu.VMEM((CHUNK, HIDDEN), jnp.float32),
            pltpu.VMEM((CHUNK, HIDDEN), jnp.float32),
            pltpu.VMEM((CHUNK, HIDDEN), jnp.float32),
            pltpu.VMEM((CHUNK, HIDDEN), jnp.float32),
            pltpu.SemaphoreType.DMA,
            pltpu.SemaphoreType.DMA,
            pltpu.SemaphoreType.DMA,
            pltpu.SemaphoreType.DMA,
        ],
    )
    def body(ix_hbm, iy_hbm, it_hbm, x_hbm, y_hbm, t_hbm, out_hbm,
             ixv, iyv, itv, bx0, by0, bt0, bx1, by1, bt1,
             sg0, sg1, ss0, ss1):
        wid = lax.axis_index("s") * 2 + lax.axis_index("c")
        w_base = wid * per_worker
        pltpu.sync_copy(ix_hbm.at[wid], ixv)
        pltpu.sync_copy(iy_hbm.at[wid], iyv)
        pltpu.sync_copy(it_hbm.at[wid], itv)

        bufs = ((bx0, by0, bt0, sg0, ss0), (bx1, by1, bt1, sg1, ss1))

        def fire(p, k):
            bx, by, bt, sg, _ = bufs[p]
            return (pltpu.async_copy(x_hbm.at[ixv.at[k]], bx, sg),
                    pltpu.async_copy(y_hbm.at[iyv.at[k]], by, sg),
                    pltpu.async_copy(t_hbm.at[itv.at[k]], bt, sg))

        gd = [None, None]
        sd = [None, None]
        gd[0] = fire(0, 0)
        for k in range(n_chunks):
            p = k & 1
            q = (k + 1) & 1
            if k + 1 < n_chunks:
                if sd[q] is not None:
                    sd[q].wait()
                gd[q] = fire(q, k + 1)
            for d in gd[p]:
                d.wait()
            bx, by, bt, _, ss = bufs[p]

            def row_step(r, _, bx=bx, by=by, bt=bt):
                for c in range(HIDDEN // LANES):
                    sl = pl.ds(c * LANES, LANES)
                    bx[r, sl] = bx[r, sl] + by[r, sl] + bt[r, sl]
                return 0

            lax.fori_loop(0, CHUNK, row_step, 0)
            sd[p] = pltpu.async_copy(
                bx, out_hbm.at[pl.ds(w_base + k * CHUNK, CHUNK)], ss)
        for d in sd:
            if d is not None:
                d.wait()

    return body(ix, iy, it, x_table, y_table, t_table)


def kernel(position_ids, x_table, y_table, time_table):
    b, s, _ = position_ids.shape
    n_tokens = b * s
    per_worker = n_tokens // N_WORKERS
    n_chunks = per_worker // CHUNK
    ids = position_ids.reshape(n_tokens, 3).astype(jnp.int32)
    ix = ids[:, 0].reshape(N_WORKERS, n_chunks, CHUNK)
    iy = ids[:, 1].reshape(N_WORKERS, n_chunks, CHUNK)
    it = ids[:, 2].reshape(N_WORKERS, n_chunks, CHUNK)
    out = _sc_embed_sum(ix, iy, it, x_table, y_table, time_table, n_tokens)
    return out.reshape(b, s, HIDDEN)
